# baseline (speedup 1.0000x reference)
import functools
import math

import jax
import jax.numpy as jnp
from jax import lax
from jax.experimental import pallas as pl
from jax.experimental.pallas import tpu as pltpu
from jax.experimental.pallas import tpu_sc as plsc

_NUM_HEADS = 16
_NUM_BUCKETS = 32
_MAX_DISTANCE = 128
_T = 2048
_LINE = 4096
_NSHIFT = 8
_FLAT = _NSHIFT * _LINE
_NC = 2
_NS = 16
_ROWS_PER_W = _NUM_HEADS * _T // (_NC * _NS)
_K = 8


def _line_tc_kernel(delta_ref, table_ref, line_ref):
    p = lax.broadcasted_iota(jnp.int32, (_NUM_BUCKETS, _FLAT), 1)
    m = p >> 12
    d = p & (_LINE - 1)
    rel = d + m - (_T - 1) + delta_ref[0]
    nb = _NUM_BUCKETS // 2
    rb = jnp.where(rel > 0, nb, 0)
    a = jnp.abs(rel)
    max_exact = nb // 2
    is_small = a < max_exact
    large = max_exact + (
        jnp.log(a.astype(jnp.float32) / max_exact)
        / math.log(_MAX_DISTANCE / max_exact)
        * (nb - max_exact)
    ).astype(jnp.int32)
    large = jnp.minimum(large, nb - 1)
    bucket = rb + jnp.where(is_small, a, large)
    b_iota = lax.broadcasted_iota(jnp.int32, (_NUM_BUCKETS, _FLAT), 0)
    onehot = (b_iota == bucket).astype(jnp.float32)
    line_ref[...] = jnp.dot(table_ref[...], onehot,
                            preferred_element_type=jnp.float32,
                            precision=lax.Precision.HIGHEST)


def _compute_line(delta, table):
    return pl.pallas_call(
        _line_tc_kernel,
        out_shape=jax.ShapeDtypeStruct((_NUM_HEADS, _FLAT), jnp.float32),
        in_specs=[
            pl.BlockSpec(memory_space=pltpu.SMEM),
            pl.BlockSpec(memory_space=pltpu.VMEM),
        ],
        out_specs=pl.BlockSpec(memory_space=pltpu.VMEM),
    )(delta, table)


_SLAB_ROWS = 8
_NSLAB = _ROWS_PER_W // _SLAB_ROWS


def _build_slab(line_v, buf, top):
    b = (_T - 1) - top

    def chunk(c, carry):
        col = c * 128
        for j in range(_SLAB_ROWS):
            mj = lax.rem(b - j, _NSHIFT)
            base = pl.multiple_of(
                (b - j - mj) + mj * _LINE + col, _NSHIFT)
            for k in range(8):
                buf[j, pl.ds(col + 16 * k, 16)] = (
                    line_v[pl.ds(base + 16 * k, 16)])
        return carry

    lax.fori_loop(0, _T // 128, chunk, 0)


def _expand_sc(line_flat):
    mesh = plsc.VectorSubcoreMesh(core_axis_name="c", subcore_axis_name="s")

    @functools.partial(
        pl.kernel,
        mesh=mesh,
        out_type=jax.ShapeDtypeStruct((_NUM_HEADS, _T, _T), jnp.float32),
        scratch_types=[
            pltpu.VMEM((_FLAT,), jnp.float32),
            pltpu.VMEM((_SLAB_ROWS, _T), jnp.float32),
            pltpu.VMEM((_SLAB_ROWS, _T), jnp.float32),
            pltpu.SemaphoreType.DMA,
            pltpu.SemaphoreType.DMA,
        ],
    )
    def k(line_hbm, out_hbm, line_v, buf_a, buf_b, sem_a, sem_b):
        wid = lax.axis_index("s") * _NC + lax.axis_index("c")
        h = wid // (_T // _ROWS_PER_W)
        row0 = (wid % (_T // _ROWS_PER_W)) * _ROWS_PER_W
        pltpu.sync_copy(
            line_hbm.at[pl.ds(pl.multiple_of(h * _FLAT, _NSHIFT), _FLAT)],
            line_v)

        def dma(buf, sem, slab):
            top = row0 + slab * _SLAB_ROWS
            return pltpu.make_async_copy(
                buf, out_hbm.at[h, pl.ds(top, _SLAB_ROWS), :], sem)

        _build_slab(line_v, buf_a, row0)
        cp = dma(buf_a, sem_a, 0)
        cp.start()

        def pair(t, carry):
            _build_slab(line_v, buf_b, row0 + (2 * t + 1) * _SLAB_ROWS)
            cpb = dma(buf_b, sem_b, 2 * t + 1)
            cpb.start()
            dma(buf_a, sem_a, 0).wait()
            _build_slab(line_v, buf_a, row0 + (2 * t + 2) * _SLAB_ROWS)
            cpa = dma(buf_a, sem_a, 2 * t + 2)
            cpa.start()
            dma(buf_b, sem_b, 0).wait()
            return carry

        lax.fori_loop(0, (_NSLAB - 2) // 2, pair, 0)

        _build_slab(line_v, buf_b, row0 + (_NSLAB - 1) * _SLAB_ROWS)
        cpb = dma(buf_b, sem_b, _NSLAB - 1)
        cpb.start()
        dma(buf_a, sem_a, 0).wait()
        dma(buf_b, sem_b, 0).wait()

    return k(line_flat)


def kernel(T_k, T_q, relative_attention_bias):
    delta = (jnp.asarray(T_k, jnp.int32)
             - jnp.asarray(T_q, jnp.int32)).reshape(1)
    line = _compute_line(delta, relative_attention_bias)
    return _expand_sc(jnp.reshape(line, (-1,)))


# speedup vs baseline: 3.1505x; 3.1505x over previous
import functools
import math

import jax
import jax.numpy as jnp
from jax import lax
from jax.experimental import pallas as pl
from jax.experimental.pallas import tpu as pltpu
from jax.experimental.pallas import tpu_sc as plsc

_NUM_HEADS = 16
_NUM_BUCKETS = 32
_MAX_DISTANCE = 128
_T = 2048
_LINE = 4096
_NROW = 8
_FLAT = _NROW * _LINE
_NCLS = 16
_NC = 2
_NS = 16
_PAIRS_PER_W = _NUM_HEADS * _NCLS // (_NC * _NS)
_SLABS_PER_CLS = _T // 128


def _line_tc_kernel(delta_ref, table_ref, line_ref):
    cls = pl.program_id(0)
    p = lax.broadcasted_iota(jnp.int32, (_NUM_BUCKETS, _FLAT), 1)
    j = p >> 12
    d = p & (_LINE - 1)
    rel = d + 8 * cls + 7 - j - (_T - 1) + delta_ref[0]
    nb = _NUM_BUCKETS // 2
    rb = jnp.where(rel > 0, nb, 0)
    a = jnp.abs(rel)
    max_exact = nb // 2
    is_small = a < max_exact
    large = max_exact + (
        jnp.log(a.astype(jnp.float32) / max_exact)
        / math.log(_MAX_DISTANCE / max_exact)
        * (nb - max_exact)
    ).astype(jnp.int32)
    large = jnp.minimum(large, nb - 1)
    bucket = rb + jnp.where(is_small, a, large)
    b_iota = lax.broadcasted_iota(jnp.int32, (_NUM_BUCKETS, _FLAT), 0)
    onehot = (b_iota == bucket).astype(jnp.float32)
    res = jnp.dot(table_ref[...], onehot,
                  preferred_element_type=jnp.float32,
                  precision=lax.Precision.HIGHEST)
    for jj in range(_NROW):
        line_ref[:, jj, :] = res[:, jj * _LINE:(jj + 1) * _LINE]


def _compute_line(delta, table):
    return pl.pallas_call(
        _line_tc_kernel,
        grid=(_NCLS,),
        out_shape=jax.ShapeDtypeStruct(
            (_NCLS * _NUM_HEADS, _NROW, _LINE), jnp.float32),
        in_specs=[
            pl.BlockSpec(memory_space=pltpu.SMEM),
            pl.BlockSpec((_NUM_HEADS, _NUM_BUCKETS), lambda c: (0, 0)),
        ],
        out_specs=pl.BlockSpec(
            (_NUM_HEADS, _NROW, _LINE), lambda c: (c, 0, 0)),
    )(delta, table)


def _expand_sc(line):
    mesh = plsc.VectorSubcoreMesh(core_axis_name="c", subcore_axis_name="s")

    @functools.partial(
        pl.kernel,
        mesh=mesh,
        out_type=jax.ShapeDtypeStruct((_NUM_HEADS, _T, _T), jnp.float32),
        scratch_types=[
            pltpu.VMEM((_NROW, _LINE), jnp.float32),
            pltpu.VMEM((_NROW, _LINE), jnp.float32),
            pltpu.SemaphoreType.DMA,
            pltpu.SemaphoreType.DMA,
            pltpu.SemaphoreType.DMA,
        ],
    )
    def k(line_hbm, out_hbm, buf_a, buf_b, sem_a, sem_b, sem_out):
        wid = lax.axis_index("s") * _NC + lax.axis_index("c")
        p0 = wid * _PAIRS_PER_W
        bufs = (buf_a, buf_b)
        sems = (sem_a, sem_b)

        def load(p):
            pair = p0 + p
            h = pair // _NCLS
            cls = lax.rem(pair, _NCLS)
            cp = pltpu.make_async_copy(
                line_hbm.at[cls * _NUM_HEADS + h], bufs[p % 2], sems[p % 2])
            cp.start()
            return cp

        def fire(p):
            pair = p0 + p
            h = pair // _NCLS
            cls = lax.rem(pair, _NCLS)
            cps = []
            for i in range(_SLABS_PER_CLS):
                top = pl.multiple_of(
                    (_T - _NROW) - 8 * cls - 128 * i, _NROW)
                cp = pltpu.make_async_copy(
                    bufs[p % 2].at[:, pl.ds(128 * i, _T)],
                    out_hbm.at[h, pl.ds(top, _NROW), :],
                    sem_out)
                cp.start()
                cps.append(cp)
            return cps

        loads = [load(0)] + [None] * (_PAIRS_PER_W - 1)
        outs = [None] * _PAIRS_PER_W
        for p in range(_PAIRS_PER_W):
            if p >= 1:
                for cp in outs[p - 1]:
                    cp.wait()
            if p + 1 < _PAIRS_PER_W:
                loads[p + 1] = load(p + 1)
            loads[p].wait()
            outs[p] = fire(p)
        for cp in outs[_PAIRS_PER_W - 1]:
            cp.wait()

    return k(line)


def kernel(T_k, T_q, relative_attention_bias):
    delta = (jnp.asarray(T_k, jnp.int32)
             - jnp.asarray(T_q, jnp.int32)).reshape(1)
    line = _compute_line(delta, relative_attention_bias)
    return _expand_sc(line)


# speedup vs baseline: 3.1612x; 1.0034x over previous
import functools
import math

import jax
import jax.numpy as jnp
from jax import lax
from jax.experimental import pallas as pl
from jax.experimental.pallas import tpu as pltpu
from jax.experimental.pallas import tpu_sc as plsc

_NUM_HEADS = 16
_NUM_BUCKETS = 32
_MAX_DISTANCE = 128
_T = 2048
_LINE = 4096
_NROW = 8
_FLAT = _NROW * _LINE
_NCLS = 16
_NC = 2
_NS = 16
_PAIRS_PER_W = _NUM_HEADS * _NCLS // (_NC * _NS)
_SLABS_PER_CLS = _T // 128


def _line_tc_kernel(delta_ref, table_ref, line_ref):
    cls = pl.program_id(0)
    p = lax.broadcasted_iota(jnp.int32, (8, _FLAT), 1)
    j = p >> 12
    d = p & (_LINE - 1)
    rel = d + 8 * cls + 7 - j - (_T - 1) + delta_ref[0]
    nb = _NUM_BUCKETS // 2
    rb = jnp.where(rel > 0, nb, 0)
    a = jnp.abs(rel)
    max_exact = nb // 2
    is_small = a < max_exact
    large = max_exact + (
        jnp.log(a.astype(jnp.float32) / max_exact)
        / math.log(_MAX_DISTANCE / max_exact)
        * (nb - max_exact)
    ).astype(jnp.int32)
    large = jnp.minimum(large, nb - 1)
    bucket = rb + jnp.where(is_small, a, large)
    b_iota = lax.broadcasted_iota(jnp.int32, (_NUM_BUCKETS, _FLAT), 0)
    onehot = (b_iota == bucket[0:1, :]).astype(jnp.float32)
    res = jnp.dot(table_ref[...], onehot,
                  preferred_element_type=jnp.float32,
                  precision=lax.Precision.HIGHEST)
    for jj in range(_NROW):
        line_ref[:, jj, :] = res[:, jj * _LINE:(jj + 1) * _LINE]


def _compute_line(delta, table):
    return pl.pallas_call(
        _line_tc_kernel,
        grid=(_NCLS,),
        out_shape=jax.ShapeDtypeStruct(
            (_NCLS * _NUM_HEADS, _NROW, _LINE), jnp.float32),
        in_specs=[
            pl.BlockSpec(memory_space=pltpu.SMEM),
            pl.BlockSpec((_NUM_HEADS, _NUM_BUCKETS), lambda c: (0, 0)),
        ],
        out_specs=pl.BlockSpec(
            (_NUM_HEADS, _NROW, _LINE), lambda c: (c, 0, 0)),
    )(delta, table)


def _expand_sc(line):
    mesh = plsc.VectorSubcoreMesh(core_axis_name="c", subcore_axis_name="s")

    @functools.partial(
        pl.kernel,
        mesh=mesh,
        out_type=jax.ShapeDtypeStruct((_NUM_HEADS, _T, _T), jnp.float32),
        scratch_types=[
            pltpu.VMEM((_NROW, _LINE), jnp.float32),
            pltpu.VMEM((_NROW, _LINE), jnp.float32),
            pltpu.VMEM((_NROW, _LINE), jnp.float32),
            pltpu.SemaphoreType.DMA,
            pltpu.SemaphoreType.DMA,
            pltpu.SemaphoreType.DMA,
            pltpu.SemaphoreType.DMA,
        ],
    )
    def k(line_hbm, out_hbm, buf_a, buf_b, buf_c,
          sem_a, sem_b, sem_c, sem_out):
        wid = lax.axis_index("s") * _NC + lax.axis_index("c")
        p0 = wid * _PAIRS_PER_W
        bufs = (buf_a, buf_b, buf_c)
        sems = (sem_a, sem_b, sem_c)

        def load(p):
            pair = p0 + p
            h = pair // _NCLS
            cls = lax.rem(pair, _NCLS)
            cp = pltpu.make_async_copy(
                line_hbm.at[cls * _NUM_HEADS + h], bufs[p % 3], sems[p % 3])
            cp.start()
            return cp

        def fire(p):
            pair = p0 + p
            h = pair // _NCLS
            cls = lax.rem(pair, _NCLS)
            cps = []
            for i in range(_SLABS_PER_CLS):
                top = pl.multiple_of(
                    (_T - _NROW) - 8 * cls - 128 * i, _NROW)
                cp = pltpu.make_async_copy(
                    bufs[p % 3].at[:, pl.ds(128 * i, _T)],
                    out_hbm.at[h, pl.ds(top, _NROW), :],
                    sem_out)
                cp.start()
                cps.append(cp)
            return cps

        loads = [load(0), load(1)] + [None] * (_PAIRS_PER_W - 2)
        outs = [None] * _PAIRS_PER_W
        for p in range(_PAIRS_PER_W):
            if p >= 2:
                for cp in outs[p - 2]:
                    cp.wait()
            if p >= 1 and p + 1 < _PAIRS_PER_W:
                loads[p + 1] = load(p + 1)
            loads[p].wait()
            outs[p] = fire(p)
        for cp in outs[_PAIRS_PER_W - 2]:
            cp.wait()
        for cp in outs[_PAIRS_PER_W - 1]:
            cp.wait()

    return k(line)


def kernel(T_k, T_q, relative_attention_bias):
    delta = (jnp.asarray(T_k, jnp.int32)
             - jnp.asarray(T_q, jnp.int32)).reshape(1)
    line = _compute_line(delta, relative_attention_bias)
    return _expand_sc(line)


# speedup vs baseline: 3.8041x; 1.2034x over previous
import functools
import math

import jax
import jax.numpy as jnp
from jax import lax
from jax.experimental import pallas as pl
from jax.experimental.pallas import tpu as pltpu
from jax.experimental.pallas import tpu_sc as plsc

_NUM_HEADS = 16
_NUM_BUCKETS = 32
_MAX_DISTANCE = 128
_T = 2048
_LINE = 4096
_NROW = 8
_FLAT = _NROW * _LINE
_NCLS = 16
_NC = 2
_NS = 16
_PAIRS_PER_W = _NUM_HEADS * _NCLS // (_NC * _NS)
_SLABS_PER_CLS = _T // 128


_MASTER_W = 4224


def _line_tc_kernel(delta_ref, table_ref, line_ref, master_ref):
    cls = pl.program_id(0)

    @pl.when(cls == 0)
    def _():
        x = lax.broadcasted_iota(jnp.int32, (8, _MASTER_W), 1)
        rel = x - (_T - 1) + delta_ref[0]
        nb = _NUM_BUCKETS // 2
        rb = jnp.where(rel > 0, nb, 0)
        a = jnp.abs(rel)
        max_exact = nb // 2
        is_small = a < max_exact
        large = max_exact + (
            jnp.log(a.astype(jnp.float32) / max_exact)
            / math.log(_MAX_DISTANCE / max_exact)
            * (nb - max_exact)
        ).astype(jnp.int32)
        large = jnp.minimum(large, nb - 1)
        bucket = rb + jnp.where(is_small, a, large)
        b_iota = lax.broadcasted_iota(
            jnp.int32, (_NUM_BUCKETS, _MASTER_W), 0)
        onehot = (b_iota == bucket[0:1, :]).astype(jnp.float32)
        master_ref[...] = jnp.dot(table_ref[...], onehot,
                                  preferred_element_type=jnp.float32,
                                  precision=lax.Precision.HIGHEST)

    for j in range(_NROW):
        shift = _MASTER_W - (8 * cls + 7 - j)
        rolled = pltpu.roll(master_ref[...], shift, 1)
        line_ref[:, j, :] = rolled[:, :_LINE]


def _compute_line(delta, table):
    return pl.pallas_call(
        _line_tc_kernel,
        grid=(_NCLS,),
        out_shape=jax.ShapeDtypeStruct(
            (_NCLS * _NUM_HEADS, _NROW, _LINE), jnp.float32),
        in_specs=[
            pl.BlockSpec(memory_space=pltpu.SMEM),
            pl.BlockSpec((_NUM_HEADS, _NUM_BUCKETS), lambda c: (0, 0)),
        ],
        out_specs=pl.BlockSpec(
            (_NUM_HEADS, _NROW, _LINE), lambda c: (c, 0, 0)),
        scratch_shapes=[pltpu.VMEM((_NUM_HEADS, _MASTER_W), jnp.float32)],
    )(delta, table)


def _expand_sc(line):
    mesh = plsc.VectorSubcoreMesh(core_axis_name="c", subcore_axis_name="s")

    @functools.partial(
        pl.kernel,
        mesh=mesh,
        out_type=jax.ShapeDtypeStruct((_NUM_HEADS, _T, _T), jnp.float32),
        scratch_types=[
            pltpu.VMEM((_NROW, _LINE), jnp.float32),
            pltpu.VMEM((_NROW, _LINE), jnp.float32),
            pltpu.VMEM((_NROW, _LINE), jnp.float32),
            pltpu.SemaphoreType.DMA,
            pltpu.SemaphoreType.DMA,
            pltpu.SemaphoreType.DMA,
            pltpu.SemaphoreType.DMA,
        ],
    )
    def k(line_hbm, out_hbm, buf_a, buf_b, buf_c,
          sem_a, sem_b, sem_c, sem_out):
        wid = lax.axis_index("s") * _NC + lax.axis_index("c")
        p0 = wid * _PAIRS_PER_W
        bufs = (buf_a, buf_b, buf_c)
        sems = (sem_a, sem_b, sem_c)

        def load(p):
            pair = p0 + p
            h = pair // _NCLS
            cls = lax.rem(pair, _NCLS)
            cp = pltpu.make_async_copy(
                line_hbm.at[cls * _NUM_HEADS + h], bufs[p % 3], sems[p % 3])
            cp.start()
            return cp

        def fire(p):
            pair = p0 + p
            h = pair // _NCLS
            cls = lax.rem(pair, _NCLS)
            cps = []
            for i in range(_SLABS_PER_CLS):
                top = pl.multiple_of(
                    (_T - _NROW) - 8 * cls - 128 * i, _NROW)
                cp = pltpu.make_async_copy(
                    bufs[p % 3].at[:, pl.ds(128 * i, _T)],
                    out_hbm.at[h, pl.ds(top, _NROW), :],
                    sem_out)
                cp.start()
                cps.append(cp)
            return cps

        loads = [load(0), load(1)] + [None] * (_PAIRS_PER_W - 2)
        outs = [None] * _PAIRS_PER_W
        for p in range(_PAIRS_PER_W):
            if p >= 2:
                for cp in outs[p - 2]:
                    cp.wait()
            if p >= 1 and p + 1 < _PAIRS_PER_W:
                loads[p + 1] = load(p + 1)
            loads[p].wait()
            outs[p] = fire(p)
        for cp in outs[_PAIRS_PER_W - 2]:
            cp.wait()
        for cp in outs[_PAIRS_PER_W - 1]:
            cp.wait()

    return k(line)


def kernel(T_k, T_q, relative_attention_bias):
    delta = (jnp.asarray(T_k, jnp.int32)
             - jnp.asarray(T_q, jnp.int32)).reshape(1)
    line = _compute_line(delta, relative_attention_bias)
    return _expand_sc(line)


# speedup vs baseline: 3.9823x; 1.0468x over previous
import functools
import math

import jax
import jax.numpy as jnp
from jax import lax
from jax.experimental import pallas as pl
from jax.experimental.pallas import tpu as pltpu
from jax.experimental.pallas import tpu_sc as plsc

_NUM_HEADS = 16
_NUM_BUCKETS = 32
_MAX_DISTANCE = 128
_T = 2048
_LINE = 4096
_NROW = 8
_FLAT = _NROW * _LINE
_NCLS = 16
_NC = 2
_NS = 16
_PAIRS_PER_W = _NUM_HEADS * _NCLS // (_NC * _NS)
_SLABS_PER_CLS = _T // 128


_MASTER_W = 4224


def _line_tc_kernel(delta_ref, table_ref, line_ref, master_ref):
    cls = pl.program_id(0)

    @pl.when(cls == 0)
    def _():
        x = lax.broadcasted_iota(jnp.int32, (8, _MASTER_W), 1)
        rel = x - (_T - 1) + delta_ref[0]
        nb = _NUM_BUCKETS // 2
        rb = jnp.where(rel > 0, nb, 0)
        a = jnp.abs(rel)
        max_exact = nb // 2
        is_small = a < max_exact
        large = max_exact + (
            jnp.log(a.astype(jnp.float32) / max_exact)
            / math.log(_MAX_DISTANCE / max_exact)
            * (nb - max_exact)
        ).astype(jnp.int32)
        large = jnp.minimum(large, nb - 1)
        bucket = rb + jnp.where(is_small, a, large)
        b_iota = lax.broadcasted_iota(
            jnp.int32, (_NUM_BUCKETS, _MASTER_W), 0)
        onehot = (b_iota == bucket[0:1, :]).astype(jnp.float32)
        master_ref[...] = jnp.dot(table_ref[...], onehot,
                                  preferred_element_type=jnp.float32,
                                  precision=lax.Precision.HIGHEST)

    r = master_ref[...]
    for j in range(_NROW - 1, -1, -1):
        line_ref[:, j, :] = r[:, :_LINE]
        r = pltpu.roll(r, _MASTER_W - 1, 1)
    master_ref[...] = r


def _compute_line(delta, table):
    return pl.pallas_call(
        _line_tc_kernel,
        grid=(_NCLS,),
        out_shape=jax.ShapeDtypeStruct(
            (_NCLS * _NUM_HEADS, _NROW, _LINE), jnp.float32),
        in_specs=[
            pl.BlockSpec(memory_space=pltpu.SMEM),
            pl.BlockSpec((_NUM_HEADS, _NUM_BUCKETS), lambda c: (0, 0)),
        ],
        out_specs=pl.BlockSpec(
            (_NUM_HEADS, _NROW, _LINE), lambda c: (c, 0, 0)),
        scratch_shapes=[pltpu.VMEM((_NUM_HEADS, _MASTER_W), jnp.float32)],
    )(delta, table)


def _expand_sc(line):
    mesh = plsc.VectorSubcoreMesh(core_axis_name="c", subcore_axis_name="s")

    @functools.partial(
        pl.kernel,
        mesh=mesh,
        out_type=jax.ShapeDtypeStruct((_NUM_HEADS, _T, _T), jnp.float32),
        scratch_types=[
            pltpu.VMEM((_NROW, _LINE), jnp.float32),
            pltpu.VMEM((_NROW, _LINE), jnp.float32),
            pltpu.VMEM((_NROW, _LINE), jnp.float32),
            pltpu.SemaphoreType.DMA,
            pltpu.SemaphoreType.DMA,
            pltpu.SemaphoreType.DMA,
            pltpu.SemaphoreType.DMA,
        ],
    )
    def k(line_hbm, out_hbm, buf_a, buf_b, buf_c,
          sem_a, sem_b, sem_c, sem_out):
        wid = lax.axis_index("s") * _NC + lax.axis_index("c")
        p0 = wid * _PAIRS_PER_W
        bufs = (buf_a, buf_b, buf_c)
        sems = (sem_a, sem_b, sem_c)

        def load(p):
            pair = p0 + p
            h = pair // _NCLS
            cls = lax.rem(pair, _NCLS)
            cp = pltpu.make_async_copy(
                line_hbm.at[cls * _NUM_HEADS + h], bufs[p % 3], sems[p % 3])
            cp.start()
            return cp

        def fire(p):
            pair = p0 + p
            h = pair // _NCLS
            cls = lax.rem(pair, _NCLS)
            cps = []
            for i in range(_SLABS_PER_CLS):
                top = pl.multiple_of(
                    (_T - _NROW) - 8 * cls - 128 * i, _NROW)
                cp = pltpu.make_async_copy(
                    bufs[p % 3].at[:, pl.ds(128 * i, _T)],
                    out_hbm.at[h, pl.ds(top, _NROW), :],
                    sem_out)
                cp.start()
                cps.append(cp)
            return cps

        loads = [load(0), load(1)] + [None] * (_PAIRS_PER_W - 2)
        outs = [None] * _PAIRS_PER_W
        for p in range(_PAIRS_PER_W):
            if p >= 2:
                for cp in outs[p - 2]:
                    cp.wait()
            if p >= 1 and p + 1 < _PAIRS_PER_W:
                loads[p + 1] = load(p + 1)
            loads[p].wait()
            outs[p] = fire(p)
        for cp in outs[_PAIRS_PER_W - 2]:
            cp.wait()
        for cp in outs[_PAIRS_PER_W - 1]:
            cp.wait()

    return k(line)


def kernel(T_k, T_q, relative_attention_bias):
    delta = (jnp.asarray(T_k, jnp.int32)
             - jnp.asarray(T_q, jnp.int32)).reshape(1)
    line = _compute_line(delta, relative_attention_bias)
    return _expand_sc(line)
